# baseline (speedup 1.0000x reference)
import jax
import jax.numpy as jnp
from jax import lax
from jax.experimental import pallas as pl
from jax.experimental.pallas import tpu as pltpu
from jax.experimental.pallas import tpu_sc as plsc

B = 16384
F = 26
FIELD_DIM = 38461
NC, NS, L = 2, 16, 16
NW = NC * NS
RPW = B // NW
IPW = RPW * F
CHUNK = 128
CPF = RPW // CHUNK
NCHUNK = IPW // CHUNK


def _body(xtw_hbm, wflat_hbm, bias_hbm, out_hbm, xbuf, gbuf, bias_v, obuf, sem):
    wid = lax.axis_index("s") * NC + lax.axis_index("c")

    pltpu.sync_copy(xtw_hbm.at[wid], xbuf)
    pltpu.sync_copy(bias_hbm, bias_v)

    def fire(k, _):
        f = k // CPF
        off = (f * FIELD_DIM).astype(jnp.int32)
        row = f
        cb = (k % CPF) * CHUNK
        for c in range(CHUNK // L):
            sl = pl.ds(cb + c * L, L)
            xbuf[row, sl] = xbuf[row, sl] + off
        pltpu.async_copy(
            wflat_hbm.at[xbuf.at[row, pl.ds(cb, CHUNK)]],
            gbuf.at[pl.ds(k * CHUNK, CHUNK)],
            sem,
        )
        return 0

    lax.fori_loop(0, NCHUNK, fire, 0)

    pltpu.make_async_copy(wflat_hbm.at[pl.ds(0, IPW)], gbuf, sem).wait()

    bias_vec = bias_v[...]

    def reduce(j, _):
        base = j * L
        vacc = bias_vec
        for f in range(F):
            vacc = vacc + gbuf[pl.ds(f * RPW + base, L)]
        obuf[pl.ds(base, L)] = 1.0 / (1.0 + jnp.exp(-vacc))
        return 0

    lax.fori_loop(0, RPW // L, reduce, 0)

    pltpu.sync_copy(obuf, out_hbm.at[pl.ds(wid * RPW, RPW)])


@jax.jit
def kernel(x, weight, bias):
    eye = jnp.eye(F, dtype=jnp.float32)
    xt = lax.dot_general(
        eye, x.astype(jnp.float32),
        dimension_numbers=(((1,), (1,)), ((), ())),
        preferred_element_type=jnp.float32,
    )
    xtw = xt.astype(jnp.int32).reshape(F, NW, RPW).swapaxes(0, 1)
    wflat = weight.reshape(-1)
    bias16 = jnp.broadcast_to(bias, (L,))

    mesh = plsc.VectorSubcoreMesh(core_axis_name="c", subcore_axis_name="s")
    run = pl.kernel(
        _body,
        out_type=jax.ShapeDtypeStruct((B,), jnp.float32),
        mesh=mesh,
        scratch_types=[
            pltpu.VMEM((F, RPW), jnp.int32),
            pltpu.VMEM((IPW,), jnp.float32),
            pltpu.VMEM((L,), jnp.float32),
            pltpu.VMEM((RPW,), jnp.float32),
            pltpu.SemaphoreType.DMA,
        ],
    )
    return run(xtw, wflat, bias16)


# speedup vs baseline: 1.0001x; 1.0001x over previous
import jax
import jax.numpy as jnp
from jax import lax
from jax.experimental import pallas as pl
from jax.experimental.pallas import tpu as pltpu
from jax.experimental.pallas import tpu_sc as plsc

B = 16384
F = 26
FIELD_DIM = 38461
NC, NS, L = 2, 16, 16
NW = NC * NS
RPW = B // NW
IPW = RPW * F
CHUNK = 128
CPF = RPW // CHUNK
NCHUNK = IPW // CHUNK


def _body(xtw_hbm, wflat_hbm, bias_hbm, out_hbm, xbuf, gbuf, bias_v, obuf, sem):
    wid = lax.axis_index("s") * NC + lax.axis_index("c")

    pltpu.sync_copy(xtw_hbm.at[wid], xbuf)
    pltpu.sync_copy(bias_hbm, bias_v)

    def fire(k, _):
        f = k // CPF
        off = (f * FIELD_DIM).astype(jnp.int32)
        row = f
        cb = (k % CPF) * CHUNK
        for c in range(CHUNK // L):
            sl = pl.ds(cb + c * L, L)
            xbuf[row, sl] = xbuf[row, sl] + off
        pltpu.async_copy(
            wflat_hbm.at[xbuf.at[row, pl.ds(cb, CHUNK)]],
            gbuf.at[pl.ds(k * CHUNK, CHUNK)],
            sem,
        )
        return 0

    lax.fori_loop(0, NCHUNK, fire, 0)

    pltpu.make_async_copy(wflat_hbm.at[pl.ds(0, IPW)], gbuf, sem).wait()

    bias_vec = bias_v[...]

    def reduce(j, _):
        base = j * L
        vacc = bias_vec
        for f in range(F):
            vacc = vacc + gbuf[pl.ds(f * RPW + base, L)]
        obuf[pl.ds(base, L)] = 1.0 / (1.0 + jnp.exp(-vacc))
        return 0

    lax.fori_loop(0, RPW // L, reduce, 0)

    pltpu.sync_copy(obuf, out_hbm.at[pl.ds(wid * RPW, RPW)])


@jax.jit
def kernel(x, weight, bias):
    eye = jnp.eye(F, dtype=jnp.float32)
    xt = lax.dot_general(
        eye, x.astype(jnp.float32),
        dimension_numbers=(((1,), (1,)), ((), ())),
        preferred_element_type=jnp.float32,
        precision=lax.Precision.HIGHEST,
    )
    xtw = xt.astype(jnp.int32).reshape(F, NW, RPW).swapaxes(0, 1)
    wflat = weight.reshape(-1)
    bias16 = jnp.broadcast_to(bias, (L,))

    mesh = plsc.VectorSubcoreMesh(core_axis_name="c", subcore_axis_name="s")
    run = pl.kernel(
        _body,
        out_type=jax.ShapeDtypeStruct((B,), jnp.float32),
        mesh=mesh,
        scratch_types=[
            pltpu.VMEM((F, RPW), jnp.int32),
            pltpu.VMEM((IPW,), jnp.float32),
            pltpu.VMEM((L,), jnp.float32),
            pltpu.VMEM((RPW,), jnp.float32),
            pltpu.SemaphoreType.DMA,
        ],
    )
    return run(xtw, wflat, bias16)
